# baseline (speedup 1.0000x reference)
import functools

import jax
import jax.numpy as jnp
from jax import lax
from jax.experimental import pallas as pl
from jax.experimental.pallas import tpu as pltpu
from jax.experimental.pallas import tpu_sc as plsc

N = 10000
E = 320000
D = 128
H = 64
EPS = 1e-7

NC = 2
NS = 16
EPW = E // NS
CB = 128
NF = EPW // CB
TAIL = EPW - NF * CB
RPS = N // NS

_f32 = jnp.float32


HC = CB // 2


def _edge_body(hv1s, ef, src, dst, out,
               i0, i1, da0, da1, db0, db1, he0, he1, hx, va, vb,
               ti, td,
               acc, ss0, ss1, sd0, sd1, sh0, sh1, sgx, ssa, ssb):
  c = lax.axis_index("c")
  s = lax.axis_index("s")
  idx = [i0, i1]
  dsta = [da0, da1]
  dstb = [db0, db1]
  hev = [he0, he1]
  ssrc = [ss0, ss1]
  sdst = [sd0, sd1]
  she = [sh0, sh1]

  zero16 = jnp.zeros((16,), _f32)

  def zrow(i, carry):
    for g in range(D // 16):
      va[i, pl.ds(g * 16, 16)] = zero16
    return carry

  lax.fori_loop(0, HC, zrow, 0)
  zbase = s * RPS
  for off in range(0, 576, HC):
    pltpu.sync_copy(va, acc.at[pl.ds(zbase + off, HC), :])
  pltpu.sync_copy(va.at[pl.ds(0, RPS - 576), :],
                  acc.at[pl.ds(zbase + 576, RPS - 576), :])
  plsc.subcore_barrier()

  ebase = s * EPW
  cH = c * H
  cN = c * N

  def issue_load(k, p):
    e0 = ebase + k * CB
    pltpu.async_copy(src.at[pl.ds(e0, CB)], idx[p], ssrc[p])
    pltpu.async_copy(dst.at[pl.ds(e0, HC)], dsta[p], sdst[p])
    pltpu.async_copy(dst.at[pl.ds(e0 + HC, HC)], dstb[p], sdst[p])
    pltpu.async_copy(ef.at[pl.ds(e0, CB), pl.ds(cH, H)], hev[p], she[p])

  def compute_half(xv, ev, vv, r0, n_edges):
    @plsc.parallel_loop(0, n_edges, step=2, unroll=4)
    def body(r):
      for u in range(2):
        for g in range(H // 16):
          cs = pl.ds(g * 16, 16)
          m = jnp.maximum(xv[r0 + r + u, cs] + ev[r0 + r + u, cs], 0.0) + EPS
          e = jnp.exp(m)
          vv[r + u, cs] = e * m
          vv[r + u, pl.ds(H + g * 16, 16)] = e

  def chunk_step(k, p, first, issue_next):
    e0 = ebase + k * CB
    q = 1 - p
    pltpu.make_async_copy(src.at[pl.ds(e0, CB)], idx[p], ssrc[p]).wait()
    for g in range(CB // 16):
      gs = pl.ds(g * 16, 16)
      idx[p][gs] = idx[p][gs] + cN
    gath = pltpu.async_copy(hv1s.at[idx[p]], hx, sgx)
    pltpu.make_async_copy(dst.at[pl.ds(e0, HC)], dsta[p], sdst[p]).wait()
    pltpu.make_async_copy(dst.at[pl.ds(e0 + HC, HC)], dstb[p],
                          sdst[p]).wait()
    pltpu.make_async_copy(ef.at[pl.ds(e0, CB), pl.ds(cH, H)], hev[p],
                          she[p]).wait()
    if not first:
      pltpu.make_async_copy(va, acc.at[dsta[q]], ssa).wait()
      pltpu.make_async_copy(vb, acc.at[dstb[q]], ssb).wait()
    if issue_next:
      issue_load(k + 1, q)
    gath.wait()
    compute_half(hx, hev[p], va, 0, HC)
    pltpu.async_copy(va, acc.at[dsta[p]], ssa, add=True)
    compute_half(hx, hev[p], vb, HC, HC)
    pltpu.async_copy(vb, acc.at[dstb[p]], ssb, add=True)

  issue_load(0, 0)
  chunk_step(0, 0, first=True, issue_next=True)

  def steady(j, carry):
    chunk_step(j * 2 + 1, 1, first=False, issue_next=True)
    chunk_step(j * 2 + 2, 0, first=False, issue_next=True)
    return carry

  lax.fori_loop(0, (NF - 2) // 2, steady, 0)
  chunk_step(NF - 1, 1, first=False, issue_next=False)
  pltpu.make_async_copy(va, acc.at[dsta[1]], ssa).wait()
  pltpu.make_async_copy(vb, acc.at[dstb[1]], ssb).wait()

  e0 = ebase + NF * CB
  pltpu.sync_copy(src.at[pl.ds(e0, TAIL)], ti)
  pltpu.sync_copy(dst.at[pl.ds(e0, TAIL)], td)
  pltpu.sync_copy(ef.at[pl.ds(e0, TAIL), pl.ds(cH, H)],
                  he0.at[pl.ds(0, TAIL), :])
  for g in range(TAIL // 16):
    gs = pl.ds(g * 16, 16)
    ti[gs] = ti[gs] + cN
  pltpu.async_copy(hv1s.at[ti], hx.at[pl.ds(0, TAIL), :], sgx).wait()
  compute_half(hx, he0, va, 0, TAIL)
  pltpu.sync_copy(va.at[pl.ds(0, TAIL), :], acc.at[td], add=True)

  plsc.subcore_barrier()
  pltpu.sync_copy(acc.at[pl.ds(s * RPS, RPS), :],
                  out.at[c, pl.ds(s * RPS, RPS), :])


_edge_pass = pl.kernel(
    _edge_body,
    out_type=jax.ShapeDtypeStruct((NC, N, D), _f32),
    mesh=plsc.VectorSubcoreMesh(core_axis_name="c", subcore_axis_name="s"),
    scratch_types=(
        [pltpu.VMEM((CB,), jnp.int32) for _ in range(2)]
        + [pltpu.VMEM((HC,), jnp.int32) for _ in range(4)]
        + [pltpu.VMEM((CB, H), _f32) for _ in range(3)]
        + [pltpu.VMEM((HC, D), _f32) for _ in range(2)]
        + [pltpu.VMEM((TAIL,), jnp.int32) for _ in range(2)]
        + [pltpu.VMEM_SHARED((N, D), _f32)]
        + [pltpu.SemaphoreType.DMA for _ in range(9)]
    ),
    compiler_params=pltpu.CompilerParams(use_tc_tiling_on_sc=False),
)


def _bn_body(x_ref, g_ref, b_ref, o_ref):
  x = x_ref[...]
  m = jnp.mean(x, axis=0, keepdims=True)
  v = jnp.mean((x - m) ** 2, axis=0, keepdims=True)
  h = (x - m) * lax.rsqrt(v + 1e-5) * g_ref[...] + b_ref[...]
  h = jnp.maximum(h, 0.0)
  o_ref[0] = h[:, :H]
  o_ref[1] = h[:, H:]


_bn = pl.pallas_call(
    _bn_body, out_shape=jax.ShapeDtypeStruct((NC, N, H), _f32))


def _agg_from(accs):
  num = jnp.concatenate([accs[0, :, :H], accs[1, :, :H]], axis=1)
  den = jnp.concatenate([accs[0, :, H:], accs[1, :, H:]], axis=1)
  return num / (den + 1e-16)


def _layer_body(accs_ref, hv_ref, w_ref, b_ref, o_ref):
  agg = _agg_from(accs_ref[...])
  o_ref[...] = (jnp.dot(agg, w_ref[...], preferred_element_type=_f32)
                + b_ref[...] + hv_ref[...])


_layer = pl.pallas_call(
    _layer_body, out_shape=jax.ShapeDtypeStruct((N, D), _f32))


def _final_body(accs_ref, hv_ref, w_ref, b_ref, wo_ref, bo_ref, o_ref):
  agg = _agg_from(accs_ref[...])
  hvn = (jnp.dot(agg, w_ref[...], preferred_element_type=_f32)
         + b_ref[...] + hv_ref[...])
  hg = jnp.mean(hvn, axis=0, keepdims=True)
  o_ref[...] = (jnp.dot(hg * hvn, wo_ref[...], preferred_element_type=_f32)
                + bo_ref[...])


_final = pl.pallas_call(
    _final_body, out_shape=jax.ShapeDtypeStruct((N, D), _f32))


@jax.jit
def kernel(edge_index, edge_feats, node_feats, bn_gamma, bn_beta, W, b,
           Wout, bout):
  src = edge_index[0].astype(jnp.int32)
  dst = edge_index[1].astype(jnp.int32)
  hv = node_feats
  out = None
  for l in range(3):
    hv1s = _bn(hv, bn_gamma[l][None], bn_beta[l][None])
    accs = _edge_pass(hv1s.reshape(NC * N, H), edge_feats, src, dst)
    if l < 2:
      hv = _layer(accs, hv, W[l], b[l][None])
    else:
      out = _final(accs, hv, W[l], b[l][None], Wout, bout[None])
  return out


# speedup vs baseline: 1.2984x; 1.2984x over previous
import functools

import jax
import jax.numpy as jnp
from jax import lax
from jax.experimental import pallas as pl
from jax.experimental.pallas import tpu as pltpu
from jax.experimental.pallas import tpu_sc as plsc

N = 10000
E = 320000
D = 128
H = 64
EPS = 1e-7

NC = 2
NS = 16
EPW = E // NS
CB = 128
NF = EPW // CB
TAIL = EPW - NF * CB
RPS = N // NS

_f32 = jnp.float32


HC = CB // 2


def _edge_body(hv1s, ef, src, dst, out,
               i0, i1, da0, da1, db0, db1, he0, he1, hx0, hx1, va, vb,
               ti, td,
               acc, ss0, ss1, sd0, sd1, sh0, sh1, sg0, sg1, ssa, ssb):
  c = lax.axis_index("c")
  s = lax.axis_index("s")
  idx = [i0, i1]
  dsta = [da0, da1]
  dstb = [db0, db1]
  hev = [he0, he1]
  hxv = [hx0, hx1]
  ssrc = [ss0, ss1]
  sdst = [sd0, sd1]
  she = [sh0, sh1]
  sgx = [sg0, sg1]

  zero16 = jnp.zeros((16,), _f32)

  def zrow(i, carry):
    for g in range(D // 16):
      va[i, pl.ds(g * 16, 16)] = zero16
    return carry

  lax.fori_loop(0, HC, zrow, 0)
  zbase = s * RPS
  for off in range(0, 576, HC):
    pltpu.sync_copy(va, acc.at[pl.ds(zbase + off, HC), :])
  pltpu.sync_copy(va.at[pl.ds(0, RPS - 576), :],
                  acc.at[pl.ds(zbase + 576, RPS - 576), :])
  plsc.subcore_barrier()

  ebase = s * EPW
  cH = c * H
  cN = c * N

  def i_src(k, p):
    e0 = ebase + k * CB
    pltpu.async_copy(src.at[pl.ds(e0, CB)], idx[p], ssrc[p])

  def w_src(k, p):
    e0 = ebase + k * CB
    pltpu.make_async_copy(src.at[pl.ds(e0, CB)], idx[p], ssrc[p]).wait()

  def i_he_dst(k, p):
    e0 = ebase + k * CB
    pltpu.async_copy(ef.at[pl.ds(e0, CB), pl.ds(cH, H)], hev[p], she[p])
    pltpu.async_copy(dst.at[pl.ds(e0, HC)], dsta[p], sdst[p])
    pltpu.async_copy(dst.at[pl.ds(e0 + HC, HC)], dstb[p], sdst[p])

  def w_he_dst(k, p):
    e0 = ebase + k * CB
    pltpu.make_async_copy(ef.at[pl.ds(e0, CB), pl.ds(cH, H)], hev[p],
                          she[p]).wait()
    pltpu.make_async_copy(dst.at[pl.ds(e0, HC)], dsta[p], sdst[p]).wait()
    pltpu.make_async_copy(dst.at[pl.ds(e0 + HC, HC)], dstb[p],
                          sdst[p]).wait()

  def adjust_and_gather(p):
    for g in range(CB // 16):
      gs = pl.ds(g * 16, 16)
      idx[p][gs] = idx[p][gs] + cN
    pltpu.async_copy(hv1s.at[idx[p]], hxv[p], sgx[p])

  def w_gather(p):
    pltpu.make_async_copy(hv1s.at[idx[p]], hxv[p], sgx[p]).wait()

  def w_scat(q):
    pltpu.make_async_copy(va, acc.at[dsta[q]], ssa).wait()
    pltpu.make_async_copy(vb, acc.at[dstb[q]], ssb).wait()

  def compute_half(xv, ev, vv, r0, n_edges):
    @plsc.parallel_loop(0, n_edges, step=2, unroll=4)
    def body(r):
      for u in range(2):
        for g in range(H // 16):
          cs = pl.ds(g * 16, 16)
          m = jnp.maximum(xv[r0 + r + u, cs] + ev[r0 + r + u, cs], 0.0) + EPS
          e = jnp.exp(m)
          vv[r + u, cs] = e * m
          vv[r + u, pl.ds(H + g * 16, 16)] = e

  def chunk_step(k, p, first, has_k1, has_k2):
    q = 1 - p
    if has_k1:
      w_src(k + 1, q)
      adjust_and_gather(q)
    w_he_dst(k, p)
    if not first:
      w_scat(q)
    if has_k1:
      i_he_dst(k + 1, q)
    w_gather(p)
    if has_k2:
      i_src(k + 2, p)
    compute_half(hxv[p], hev[p], va, 0, HC)
    pltpu.async_copy(va, acc.at[dsta[p]], ssa, add=True)
    compute_half(hxv[p], hev[p], vb, HC, HC)
    pltpu.async_copy(vb, acc.at[dstb[p]], ssb, add=True)

  i_src(0, 0)
  w_src(0, 0)
  adjust_and_gather(0)
  i_he_dst(0, 0)
  i_src(1, 1)
  chunk_step(0, 0, first=True, has_k1=True, has_k2=True)

  def steady(j, carry):
    chunk_step(j * 2 + 1, 1, False, True, True)
    chunk_step(j * 2 + 2, 0, False, True, True)
    return carry

  lax.fori_loop(0, (NF - 4) // 2, steady, 0)
  chunk_step(NF - 3, 1, False, True, True)
  chunk_step(NF - 2, 0, False, True, False)
  chunk_step(NF - 1, 1, False, False, False)
  pltpu.make_async_copy(va, acc.at[dsta[1]], ssa).wait()
  pltpu.make_async_copy(vb, acc.at[dstb[1]], ssb).wait()

  e0 = ebase + NF * CB
  pltpu.sync_copy(src.at[pl.ds(e0, TAIL)], ti)
  pltpu.sync_copy(dst.at[pl.ds(e0, TAIL)], td)
  pltpu.sync_copy(ef.at[pl.ds(e0, TAIL), pl.ds(cH, H)],
                  he0.at[pl.ds(0, TAIL), :])
  for g in range(TAIL // 16):
    gs = pl.ds(g * 16, 16)
    ti[gs] = ti[gs] + cN
  pltpu.async_copy(hv1s.at[ti], hx0.at[pl.ds(0, TAIL), :], sg0).wait()
  compute_half(hx0, he0, va, 0, TAIL)
  pltpu.sync_copy(va.at[pl.ds(0, TAIL), :], acc.at[td], add=True)

  plsc.subcore_barrier()
  pltpu.sync_copy(acc.at[pl.ds(s * RPS, RPS), :],
                  out.at[c, pl.ds(s * RPS, RPS), :])


_edge_pass = pl.kernel(
    _edge_body,
    out_type=jax.ShapeDtypeStruct((NC, N, D), _f32),
    mesh=plsc.VectorSubcoreMesh(core_axis_name="c", subcore_axis_name="s"),
    scratch_types=(
        [pltpu.VMEM((CB,), jnp.int32) for _ in range(2)]
        + [pltpu.VMEM((HC,), jnp.int32) for _ in range(4)]
        + [pltpu.VMEM((CB, H), _f32) for _ in range(4)]
        + [pltpu.VMEM((HC, D), _f32) for _ in range(2)]
        + [pltpu.VMEM((TAIL,), jnp.int32) for _ in range(2)]
        + [pltpu.VMEM_SHARED((N, D), _f32)]
        + [pltpu.SemaphoreType.DMA for _ in range(10)]
    ),
    compiler_params=pltpu.CompilerParams(use_tc_tiling_on_sc=False),
)


def _bn_body(x_ref, g_ref, b_ref, o_ref):
  x = x_ref[...]
  m = jnp.mean(x, axis=0, keepdims=True)
  v = jnp.mean((x - m) ** 2, axis=0, keepdims=True)
  h = (x - m) * lax.rsqrt(v + 1e-5) * g_ref[...] + b_ref[...]
  h = jnp.maximum(h, 0.0)
  o_ref[0] = h[:, :H]
  o_ref[1] = h[:, H:]


_bn = pl.pallas_call(
    _bn_body, out_shape=jax.ShapeDtypeStruct((NC, N, H), _f32))


def _agg_from(accs):
  num = jnp.concatenate([accs[0, :, :H], accs[1, :, :H]], axis=1)
  den = jnp.concatenate([accs[0, :, H:], accs[1, :, H:]], axis=1)
  return num / (den + 1e-16)


def _layer_body(accs_ref, hv_ref, w_ref, b_ref, o_ref):
  agg = _agg_from(accs_ref[...])
  o_ref[...] = (jnp.dot(agg, w_ref[...], preferred_element_type=_f32)
                + b_ref[...] + hv_ref[...])


_layer = pl.pallas_call(
    _layer_body, out_shape=jax.ShapeDtypeStruct((N, D), _f32))


def _final_body(accs_ref, hv_ref, w_ref, b_ref, wo_ref, bo_ref, o_ref):
  agg = _agg_from(accs_ref[...])
  hvn = (jnp.dot(agg, w_ref[...], preferred_element_type=_f32)
         + b_ref[...] + hv_ref[...])
  hg = jnp.mean(hvn, axis=0, keepdims=True)
  o_ref[...] = (jnp.dot(hg * hvn, wo_ref[...], preferred_element_type=_f32)
                + bo_ref[...])


_final = pl.pallas_call(
    _final_body, out_shape=jax.ShapeDtypeStruct((N, D), _f32))


@jax.jit
def kernel(edge_index, edge_feats, node_feats, bn_gamma, bn_beta, W, b,
           Wout, bout):
  src = edge_index[0].astype(jnp.int32)
  dst = edge_index[1].astype(jnp.int32)
  hv = node_feats
  out = None
  for l in range(3):
    hv1s = _bn(hv, bn_gamma[l][None], bn_beta[l][None])
    accs = _edge_pass(hv1s.reshape(NC * N, H), edge_feats, src, dst)
    if l < 2:
      hv = _layer(accs, hv, W[l], b[l][None])
    else:
      out = _final(accs, hv, W[l], b[l][None], Wout, bout[None])
  return out


# speedup vs baseline: 1.3065x; 1.0062x over previous
import functools

import jax
import jax.numpy as jnp
from jax import lax
from jax.experimental import pallas as pl
from jax.experimental.pallas import tpu as pltpu
from jax.experimental.pallas import tpu_sc as plsc

N = 10000
E = 320000
D = 128
H = 64
EPS = 1e-7

NC = 2
NS = 16
EPW = E // NS
CB = 128
NF = EPW // CB
TAIL = EPW - NF * CB
RPS = N // NS

_f32 = jnp.float32


HC = CB // 2


def _edge_body(hv1s, ef, src, dst, out,
               i0, i1, da0, da1, db0, db1, he0, he1, hx0, hx1, va, vb,
               ti, td,
               acc, ss0, ss1, sd0, sd1, sh0, sh1, sg0, sg1, ssa, ssb):
  c = lax.axis_index("c")
  s = lax.axis_index("s")
  idx = [i0, i1]
  dsta = [da0, da1]
  dstb = [db0, db1]
  hev = [he0, he1]
  hxv = [hx0, hx1]
  ssrc = [ss0, ss1]
  sdst = [sd0, sd1]
  she = [sh0, sh1]
  sgx = [sg0, sg1]

  zero16 = jnp.zeros((16,), _f32)

  def zrow(i, carry):
    for g in range(D // 16):
      va[i, pl.ds(g * 16, 16)] = zero16
    return carry

  lax.fori_loop(0, HC, zrow, 0)
  zbase = s * RPS
  for off in range(0, 576, HC):
    pltpu.sync_copy(va, acc.at[pl.ds(zbase + off, HC), :])
  pltpu.sync_copy(va.at[pl.ds(0, RPS - 576), :],
                  acc.at[pl.ds(zbase + 576, RPS - 576), :])
  plsc.subcore_barrier()

  ebase = s * EPW
  cH = c * H
  cN = c * N

  def i_src(k, p):
    e0 = ebase + k * CB
    pltpu.async_copy(src.at[pl.ds(e0, CB)], idx[p], ssrc[p])

  def w_src(k, p):
    e0 = ebase + k * CB
    pltpu.make_async_copy(src.at[pl.ds(e0, CB)], idx[p], ssrc[p]).wait()

  def i_he_dst(k, p):
    e0 = ebase + k * CB
    pltpu.async_copy(ef.at[pl.ds(e0, CB), pl.ds(cH, H)], hev[p], she[p])
    pltpu.async_copy(dst.at[pl.ds(e0, HC)], dsta[p], sdst[p])
    pltpu.async_copy(dst.at[pl.ds(e0 + HC, HC)], dstb[p], sdst[p])

  def w_he_dst(k, p):
    e0 = ebase + k * CB
    pltpu.make_async_copy(ef.at[pl.ds(e0, CB), pl.ds(cH, H)], hev[p],
                          she[p]).wait()
    pltpu.make_async_copy(dst.at[pl.ds(e0, HC)], dsta[p], sdst[p]).wait()
    pltpu.make_async_copy(dst.at[pl.ds(e0 + HC, HC)], dstb[p],
                          sdst[p]).wait()

  def adjust_and_gather(p):
    for g in range(CB // 16):
      gs = pl.ds(g * 16, 16)
      idx[p][gs] = idx[p][gs] + cN
    pltpu.async_copy(hv1s.at[idx[p]], hxv[p], sgx[p])

  def w_gather(p):
    pltpu.make_async_copy(hv1s.at[idx[p]], hxv[p], sgx[p]).wait()

  def w_scat(q):
    pltpu.make_async_copy(va, acc.at[dsta[q]], ssa).wait()
    pltpu.make_async_copy(vb, acc.at[dstb[q]], ssb).wait()

  def compute_half(xv, ev, vv, r0, n_edges):
    @plsc.parallel_loop(0, n_edges, step=2, unroll=4)
    def body(r):
      for u in range(2):
        for g in range(H // 16):
          cs = pl.ds(g * 16, 16)
          m = jnp.maximum(xv[r0 + r + u, cs] + ev[r0 + r + u, cs], 0.0) + EPS
          e = jnp.exp(m)
          vv[r + u, cs] = e * m
          vv[r + u, pl.ds(H + g * 16, 16)] = e

  def chunk_step(k, p, first, has_k1, has_k2):
    q = 1 - p
    if has_k1:
      w_src(k + 1, q)
      adjust_and_gather(q)
    w_he_dst(k, p)
    if not first:
      w_scat(q)
    if has_k1:
      i_he_dst(k + 1, q)
    w_gather(p)
    if has_k2:
      i_src(k + 2, p)
    compute_half(hxv[p], hev[p], va, 0, HC)
    pltpu.async_copy(va, acc.at[dsta[p]], ssa, add=True)
    compute_half(hxv[p], hev[p], vb, HC, HC)
    pltpu.async_copy(vb, acc.at[dstb[p]], ssb, add=True)

  i_src(0, 0)
  w_src(0, 0)
  adjust_and_gather(0)
  i_he_dst(0, 0)
  i_src(1, 1)
  chunk_step(0, 0, first=True, has_k1=True, has_k2=True)

  def steady(j, carry):
    chunk_step(j * 2 + 1, 1, False, True, True)
    chunk_step(j * 2 + 2, 0, False, True, True)
    return carry

  lax.fori_loop(0, (NF - 4) // 2, steady, 0)
  chunk_step(NF - 3, 1, False, True, True)
  chunk_step(NF - 2, 0, False, True, False)
  chunk_step(NF - 1, 1, False, False, False)
  pltpu.make_async_copy(va, acc.at[dsta[1]], ssa).wait()
  pltpu.make_async_copy(vb, acc.at[dstb[1]], ssb).wait()

  e0 = ebase + NF * CB
  pltpu.sync_copy(src.at[pl.ds(e0, TAIL)], ti)
  pltpu.sync_copy(dst.at[pl.ds(e0, TAIL)], td)
  pltpu.sync_copy(ef.at[pl.ds(e0, TAIL), pl.ds(cH, H)],
                  he0.at[pl.ds(0, TAIL), :])
  for g in range(TAIL // 16):
    gs = pl.ds(g * 16, 16)
    ti[gs] = ti[gs] + cN
  pltpu.async_copy(hv1s.at[ti], hx0.at[pl.ds(0, TAIL), :], sg0).wait()
  compute_half(hx0, he0, va, 0, TAIL)
  pltpu.sync_copy(va.at[pl.ds(0, TAIL), :], acc.at[td], add=True)

  plsc.subcore_barrier()
  pltpu.sync_copy(acc.at[pl.ds(s * RPS, RPS), :],
                  out.at[c, pl.ds(s * RPS, RPS), :])


_edge_pass = pl.kernel(
    _edge_body,
    out_type=jax.ShapeDtypeStruct((NC, N, D), _f32),
    mesh=plsc.VectorSubcoreMesh(core_axis_name="c", subcore_axis_name="s"),
    scratch_types=(
        [pltpu.VMEM((CB,), jnp.int32) for _ in range(2)]
        + [pltpu.VMEM((HC,), jnp.int32) for _ in range(4)]
        + [pltpu.VMEM((CB, H), _f32) for _ in range(4)]
        + [pltpu.VMEM((HC, D), _f32) for _ in range(2)]
        + [pltpu.VMEM((TAIL,), jnp.int32) for _ in range(2)]
        + [pltpu.VMEM_SHARED((N, D), _f32)]
        + [pltpu.SemaphoreType.DMA for _ in range(10)]
    ),
    compiler_params=pltpu.CompilerParams(use_tc_tiling_on_sc=False),
)


def _bn_body(x_ref, g_ref, b_ref, o_ref):
  x = x_ref[...]
  m = jnp.mean(x, axis=0, keepdims=True)
  v = jnp.mean((x - m) ** 2, axis=0, keepdims=True)
  h = (x - m) * lax.rsqrt(v + 1e-5) * g_ref[...] + b_ref[...]
  h = jnp.maximum(h, 0.0)
  o_ref[0] = h[:, :H]
  o_ref[1] = h[:, H:]


_bn = pl.pallas_call(
    _bn_body, out_shape=jax.ShapeDtypeStruct((NC, N, H), _f32))


def _agg_from(accs):
  num = jnp.concatenate([accs[0, :, :H], accs[1, :, :H]], axis=1)
  den = jnp.concatenate([accs[0, :, H:], accs[1, :, H:]], axis=1)
  return num / (den + 1e-16)


def _layer_body(accs_ref, hv_ref, w_ref, b_ref, g_ref, be_ref,
                o_hv, o_h1):
  agg = _agg_from(accs_ref[...])
  hvn = (jnp.dot(agg, w_ref[...], preferred_element_type=_f32)
         + b_ref[...] + hv_ref[...])
  o_hv[...] = hvn
  m = jnp.mean(hvn, axis=0, keepdims=True)
  v = jnp.mean((hvn - m) ** 2, axis=0, keepdims=True)
  h = (hvn - m) * lax.rsqrt(v + 1e-5) * g_ref[...] + be_ref[...]
  h = jnp.maximum(h, 0.0)
  o_h1[0] = h[:, :H]
  o_h1[1] = h[:, H:]


_layer = pl.pallas_call(
    _layer_body,
    out_shape=(jax.ShapeDtypeStruct((N, D), _f32),
               jax.ShapeDtypeStruct((NC, N, H), _f32)))


def _final_body(accs_ref, hv_ref, w_ref, b_ref, wo_ref, bo_ref, o_ref):
  agg = _agg_from(accs_ref[...])
  hvn = (jnp.dot(agg, w_ref[...], preferred_element_type=_f32)
         + b_ref[...] + hv_ref[...])
  hg = jnp.mean(hvn, axis=0, keepdims=True)
  o_ref[...] = (jnp.dot(hg * hvn, wo_ref[...], preferred_element_type=_f32)
                + bo_ref[...])


_final = pl.pallas_call(
    _final_body, out_shape=jax.ShapeDtypeStruct((N, D), _f32))


@jax.jit
def kernel(edge_index, edge_feats, node_feats, bn_gamma, bn_beta, W, b,
           Wout, bout):
  src = edge_index[0].astype(jnp.int32)
  dst = edge_index[1].astype(jnp.int32)
  hv = node_feats
  hv1s = _bn(hv, bn_gamma[0][None], bn_beta[0][None])
  for l in range(2):
    accs = _edge_pass(hv1s.reshape(NC * N, H), edge_feats, src, dst)
    hv, hv1s = _layer(accs, hv, W[l], b[l][None],
                      bn_gamma[l + 1][None], bn_beta[l + 1][None])
  accs = _edge_pass(hv1s.reshape(NC * N, H), edge_feats, src, dst)
  return _final(accs, hv, W[2], b[2][None], Wout, bout[None])
